# baseline (speedup 1.0000x reference)
import jax
import jax.numpy as jnp
from jax.experimental import pallas as pl

N = 10000
E = 160000
Z = 4
F = 128
NB = 8
R_MAX = 5.0
P = 5.0
NG = 1
AVG_NEIGH = 16.0
NLAYERS = 2


def _bessel(r):
    n = jnp.arange(1, NB + 1, dtype=jnp.float32)
    rr = jnp.clip(r, 1e-6, None)
    return jnp.sqrt(2.0 / R_MAX) * jnp.sin(n[None, :] * jnp.pi * rr[:, None] / R_MAX) / rr[:, None]


def _poly_cutoff(r):
    x = r / R_MAX
    p = P
    env = 1.0 - ((p + 1.0) * (p + 2.0) / 2.0) * x ** p + p * (p + 2.0) * x ** (p + 1.0) - (p * (p + 1.0) / 2.0) * x ** (p + 2.0)
    return jnp.where(x < 1.0, env, 0.0)


def _sph(vec):
    nrm = jnp.clip(jnp.linalg.norm(vec, axis=-1, keepdims=True), 1e-9, None)
    u = vec / nrm
    s3 = jnp.sqrt(3.0)
    return jnp.stack([jnp.ones_like(u[:, 0]), s3 * u[:, 1], s3 * u[:, 2], s3 * u[:, 0]], axis=-1)


def _node_update_block(scal_ref, vsq_ref, wprod_ref, out_ref):
    x = scal_ref[...] + vsq_ref[...]
    act = x * jax.nn.sigmoid(x)
    out_ref[...] = act @ wprod_ref[...]


def _node_update(scal, vsq, w_prod):
    n = scal.shape[0]
    blk = 2000
    return pl.pallas_call(
        _node_update_block,
        grid=(n // blk,),
        in_specs=[
            pl.BlockSpec((blk, F), lambda i: (i, 0)),
            pl.BlockSpec((blk, F), lambda i: (i, 0)),
            pl.BlockSpec((F, F), lambda i: (0, 0)),
        ],
        out_specs=pl.BlockSpec((blk, F), lambda i: (i, 0)),
        out_shape=jax.ShapeDtypeStruct((n, F), jnp.float32),
    )(scal, vsq, w_prod)


def kernel(positions, node_attrs, charges, shifts, W_E0, W_embed, W_msg, W1, W2, W3, W4, W_prod, W_read, W_q, edge_index, batch, ptr):
    n_nodes = positions.shape[0]
    sender = edge_index[0]
    recv = edge_index[1]
    node_e0 = node_attrs @ W_E0
    e_total = jax.ops.segment_sum(node_e0, batch, num_segments=NG)
    feats = node_attrs @ W_embed
    vec = positions[recv] - positions[sender] + shifts
    lengths = jnp.linalg.norm(vec, axis=-1)
    sh = _sph(vec)
    edge_feats = _bessel(lengths) * _poly_cutoff(lengths)[:, None]
    q = jnp.zeros((n_nodes,), dtype=positions.dtype)
    for l in range(NLAYERS):
        h = jax.nn.silu(edge_feats @ W1[l])
        h = jax.nn.silu(h @ W2[l])
        h = jax.nn.silu(h @ W3[l])
        tp_w = h @ W4[l]
        m = (feats @ W_msg[l])[sender] * tp_w
        msg = sh[:, :, None] * m[:, None, :]
        agg = jax.ops.segment_sum(msg, recv, num_segments=n_nodes) / AVG_NEIGH
        scal = agg[:, 0, :]
        vsq = jnp.sum(agg[:, 1:, :] ** 2, axis=1)
        feats = _node_update(scal, vsq, W_prod[l])
        e_total = e_total + jax.ops.segment_sum(feats @ W_read[l], batch, num_segments=NG)
        q = q + feats @ W_q[l]
    total_q = charges + q
    dipole = jax.ops.segment_sum(positions * total_q[:, None], batch, num_segments=NG)
    e_coul = 0.5 * jax.ops.segment_sum(total_q ** 2, batch, num_segments=NG)
    return e_total + e_coul + 1e-6 * jnp.sum(dipole ** 2, axis=-1)


# speedup vs baseline: 16.9534x; 16.9534x over previous
import functools

import jax
import jax.numpy as jnp
from jax import lax
from jax.experimental import pallas as pl
from jax.experimental.pallas import tpu as pltpu
from jax.experimental.pallas import tpu_sc as plsc

N = 10000
E = 160000
Z = 4
F = 128
NB = 8
R_MAX = 5.0
P = 5.0
NG = 1
AVG_NEIGH = 16.0
NLAYERS = 2

NC = 2
NS = 16
NW = NC * NS
E_PAD = 163840
EPT = E_PAD // NW


def _bessel(r):
    n = jnp.arange(1, NB + 1, dtype=jnp.float32)
    rr = jnp.clip(r, 1e-6, None)
    return jnp.sqrt(2.0 / R_MAX) * jnp.sin(n[None, :] * jnp.pi * rr[:, None] / R_MAX) / rr[:, None]


def _poly_cutoff(r):
    x = r / R_MAX
    p = P
    env = 1.0 - ((p + 1.0) * (p + 2.0) / 2.0) * x ** p + p * (p + 2.0) * x ** (p + 1.0) - (p * (p + 1.0) / 2.0) * x ** (p + 2.0)
    return jnp.where(x < 1.0, env, 0.0)


def _sph(vec):
    nrm = jnp.clip(jnp.linalg.norm(vec, axis=-1, keepdims=True), 1e-9, None)
    u = vec / nrm
    s3 = jnp.sqrt(3.0)
    return jnp.stack([jnp.ones_like(u[:, 0]), s3 * u[:, 1], s3 * u[:, 2], s3 * u[:, 0]], axis=-1)


def _s1_body(px, py, pz, es, er, dx, dy, dz,
             pxv, pyv, pzv, esv, erv, dxv, dyv, dzv):
    wid = lax.axis_index("s") * NC + lax.axis_index("c")
    base = wid * EPT
    pltpu.sync_copy(px, pxv)
    pltpu.sync_copy(py, pyv)
    pltpu.sync_copy(pz, pzv)
    pltpu.sync_copy(es.at[pl.ds(base, EPT)], esv)
    pltpu.sync_copy(er.at[pl.ds(base, EPT)], erv)

    def step(i, carry):
        o = i * 16
        s_idx = esv[pl.ds(o, 16)]
        r_idx = erv[pl.ds(o, 16)]
        dxv[pl.ds(o, 16)] = (plsc.load_gather(pxv, [r_idx])
                             - plsc.load_gather(pxv, [s_idx]))
        dyv[pl.ds(o, 16)] = (plsc.load_gather(pyv, [r_idx])
                             - plsc.load_gather(pyv, [s_idx]))
        dzv[pl.ds(o, 16)] = (plsc.load_gather(pzv, [r_idx])
                             - plsc.load_gather(pzv, [s_idx]))
        return carry

    lax.fori_loop(0, EPT // 16, step, 0)
    pltpu.sync_copy(dxv, dx.at[pl.ds(base, EPT)])
    pltpu.sync_copy(dyv, dy.at[pl.ds(base, EPT)])
    pltpu.sync_copy(dzv, dz.at[pl.ds(base, EPT)])


def _edge_vectors(px, py, pz, es, er):
    mesh = plsc.VectorSubcoreMesh(core_axis_name="c", subcore_axis_name="s")
    f32 = jnp.float32
    out = jax.ShapeDtypeStruct((E_PAD,), f32)
    k = pl.kernel(
        _s1_body,
        out_type=(out, out, out),
        mesh=mesh,
        compiler_params=pltpu.CompilerParams(needs_layout_passes=False),
        scratch_types=(
            pltpu.VMEM((N,), f32),
            pltpu.VMEM((N,), f32),
            pltpu.VMEM((N,), f32),
            pltpu.VMEM((EPT,), jnp.int32),
            pltpu.VMEM((EPT,), jnp.int32),
            pltpu.VMEM((EPT,), f32),
            pltpu.VMEM((EPT,), f32),
            pltpu.VMEM((EPT,), f32),
        ),
    )
    return k(px, py, pz, es, er)


N_ACC = 10112
EPT2 = E_PAD // NS
WINS = EPT2 // 128
RPT = N_ACC // NS


def _s2_body(pre, tpw, shf, es, ersc, zrows, out,
             idxv, ridxv, shv, tpwv, gv, acc, sem):
    c = lax.axis_index("c")
    s = lax.axis_index("s")
    ebase = s * EPT2
    rbase = s * RPT
    for p in range(2):
        ch = 2 * p + c
        pltpu.sync_copy(zrows, acc.at[pl.ds(rbase, RPT)])
        plsc.subcore_barrier()

        def win(w, carry):
            e0 = ebase + w * 128
            pltpu.sync_copy(es.at[pl.ds(e0, 128)], idxv)
            pltpu.sync_copy(ersc.at[pl.ds(e0, 128)], ridxv)
            pltpu.sync_copy(shf.at[ch, pl.ds(e0, 128)], shv.at[pl.ds(0, 128)])
            pltpu.sync_copy(tpw.at[pl.ds(e0, 128)], tpwv)
            pltpu.async_copy(pre.at[idxv], gv, sem).wait()

            def edge(e, cc):
                sc = shv[pl.ds(e, 16)][0]
                for k in range(8):
                    sl = pl.ds(k * 16, 16)
                    gv[e, sl] = gv[e, sl] * tpwv[e, sl] * sc
                return cc

            lax.fori_loop(0, 128, edge, 0)
            pltpu.sync_copy(gv, acc.at[ridxv], add=True)
            return carry

        lax.fori_loop(0, WINS, win, 0)
        plsc.subcore_barrier()
        for r0 in range(0, RPT, 128):
            rn = min(128, RPT - r0)
            pltpu.sync_copy(acc.at[pl.ds(rbase + r0, rn)], gv.at[pl.ds(0, rn)])
            pltpu.sync_copy(gv.at[pl.ds(0, rn)], out.at[ch, pl.ds(rbase + r0, rn)])
        plsc.subcore_barrier()


def _aggregate(pre, tpw, shf, es, ersc, zrows):
    mesh = plsc.VectorSubcoreMesh(core_axis_name="c", subcore_axis_name="s")
    f32 = jnp.float32
    k = pl.kernel(
        _s2_body,
        out_type=jax.ShapeDtypeStruct((4, N_ACC, F), f32),
        mesh=mesh,
        compiler_params=pltpu.CompilerParams(needs_layout_passes=False),
        scratch_types=(
            pltpu.VMEM((128,), jnp.int32),
            pltpu.VMEM((128,), jnp.int32),
            pltpu.VMEM((144,), f32),
            pltpu.VMEM((128, F), f32),
            pltpu.VMEM((128, F), f32),
            pltpu.VMEM_SHARED((N_ACC, F), f32),
            pltpu.SemaphoreType.DMA,
        ),
    )
    return k(pre, tpw, shf, es, ersc, zrows)


def kernel(positions, node_attrs, charges, shifts, W_E0, W_embed, W_msg, W1, W2, W3, W4, W_prod, W_read, W_q, edge_index, batch, ptr):
    n_nodes = positions.shape[0]
    sender = edge_index[0]
    recv = edge_index[1]

    px = positions[:, 0]
    py = positions[:, 1]
    pz = positions[:, 2]
    padidx = (jnp.arange(E, E_PAD, dtype=jnp.int32) % N).astype(jnp.int32)
    es_pad = jnp.concatenate([sender.astype(jnp.int32), padidx])
    er_pad = jnp.concatenate([recv.astype(jnp.int32), padidx])
    ersc = jnp.concatenate([
        recv.astype(jnp.int32),
        N + (jnp.arange(E, E_PAD, dtype=jnp.int32) % 16),
    ])
    dx, dy, dz = _edge_vectors(px, py, pz, es_pad, er_pad)
    vec = jnp.stack([dx, dy, dz], axis=-1)

    node_e0 = node_attrs @ W_E0
    e_total = jnp.sum(node_e0)[None]
    feats = node_attrs @ W_embed
    lengths = jnp.linalg.norm(vec, axis=-1)
    sh = _sph(vec)
    sh_full = jnp.concatenate(
        [jnp.ones((1, E_PAD), jnp.float32), sh[:, 1:].T], axis=0)
    zrows = jnp.zeros((RPT, F), jnp.float32)
    edge_feats = _bessel(lengths) * _poly_cutoff(lengths)[:, None]
    q = jnp.zeros((n_nodes,), dtype=positions.dtype)
    for l in range(NLAYERS):
        h = jax.nn.silu(edge_feats @ W1[l])
        h = jax.nn.silu(h @ W2[l])
        h = jax.nn.silu(h @ W3[l])
        tp_w = h @ W4[l]
        pre = feats @ W_msg[l]
        acc4 = _aggregate(pre, tp_w, sh_full, es_pad, ersc, zrows)
        scal = acc4[0, :N] / AVG_NEIGH
        vsq = (acc4[1, :N] ** 2 + acc4[2, :N] ** 2 + acc4[3, :N] ** 2) / AVG_NEIGH ** 2
        feats = jax.nn.silu(scal + vsq) @ W_prod[l]
        e_total = e_total + jnp.sum(feats @ W_read[l])
        q = q + feats @ W_q[l]
    total_q = charges + q
    dipole = jnp.sum(positions * total_q[:, None], axis=0)[None, :]
    e_coul = 0.5 * jnp.sum(total_q ** 2)[None]
    return e_total + e_coul + 1e-6 * jnp.sum(dipole ** 2, axis=-1)
